# baseline (speedup 1.0000x reference)
import jax
import jax.numpy as jnp
import numpy as np
from jax import lax
from jax.experimental import pallas as pl
from jax.experimental.pallas import tpu as pltpu

B = 2048
D = 1024
E = 8
A = 16
H = 64
HC = 2 * H
OC = 32
EH = E * H
BM = 256


def _body(obs_ref, idx_ref, wa1_ref, ba1_ref, wa2_ref, ba2_ref,
          wc1_ref, bc1_ref, wc2_ref, bc2_ref, out_ref,
          wta_scr, wtc_scr, b1_scr, w2_scr, b2_scr):
    @pl.when(pl.program_id(0) == 0)
    def _prep():
        w2_scr[...] = jnp.zeros((E, HC, OC), jnp.float32)
        b2_scr[...] = jnp.zeros((E, OC), jnp.float32)
        for e in range(E):
            wta_scr[:, e * H:(e + 1) * H] = wa1_ref[e].astype(jnp.bfloat16)
            wtc_scr[:, e * H:(e + 1) * H] = wc1_ref[e].astype(jnp.bfloat16)
            b1_scr[e:e + 1, :H] = ba1_ref[e:e + 1]
            b1_scr[e:e + 1, H:] = bc1_ref[e:e + 1]
            w2_scr[e, :H, :A] = wa2_ref[e]
            w2_scr[e, H:, A:A + 1] = wc2_ref[e]
            b2_scr[e:e + 1, :A] = ba2_ref[e:e + 1]
            b2_scr[e:e + 1, A:A + 1] = bc2_ref[e:e + 1]

    x = obs_ref[...].astype(jnp.bfloat16)
    idx = idx_ref[...]
    ha_all = lax.dot_general(x, wta_scr[...], (((1,), (0,)), ((), ())),
                             preferred_element_type=jnp.float32)
    hc_all = lax.dot_general(x, wtc_scr[...], (((1,), (0,)), ((), ())),
                             preferred_element_type=jnp.float32)
    pre = jnp.zeros((BM, HC), jnp.float32)
    for e in range(E):
        he = jnp.concatenate(
            [ha_all[:, e * H:(e + 1) * H], hc_all[:, e * H:(e + 1) * H]],
            axis=1)
        pre = jnp.where(idx == e, he, pre)
    b1 = jnp.zeros((BM, HC), jnp.float32)
    for e in range(E):
        b1 = jnp.where(idx == e, b1_scr[e][None, :], b1)
    h = jnp.tanh(pre + b1)
    acc = jnp.zeros((BM, OC), jnp.float32)
    for e in range(E):
        oe = lax.dot_general(h, w2_scr[e], (((1,), (0,)), ((), ())),
                             preferred_element_type=jnp.float32)
        acc = jnp.where(idx == e, oe + b2_scr[e][None, :], acc)
    out_ref[...] = acc


@jax.jit
def kernel(obs, idxs, Wa1, ba1, Wa2, ba2, Wc1, bc1, Wc2, bc2):
    idx2 = idxs.astype(jnp.int32).reshape(B, 1)

    def full(shape):
        return pl.BlockSpec(shape, lambda i: tuple(0 for _ in shape))

    out = pl.pallas_call(
        _body,
        grid=(B // BM,),
        in_specs=[
            pl.BlockSpec((BM, D), lambda i: (i, 0)),
            pl.BlockSpec((BM, 1), lambda i: (i, 0)),
            full((E, D, H)),
            full((E, H)),
            full((E, H, A)),
            full((E, A)),
            full((E, D, H)),
            full((E, H)),
            full((E, H, 1)),
            full((E, 1)),
        ],
        out_specs=pl.BlockSpec((BM, OC), lambda i: (i, 0)),
        out_shape=jax.ShapeDtypeStruct((B, OC), jnp.float32),
        scratch_shapes=[
            pltpu.VMEM((D, EH), jnp.bfloat16),
            pltpu.VMEM((D, EH), jnp.bfloat16),
            pltpu.VMEM((E, HC), jnp.float32),
            pltpu.VMEM((E, HC, OC), jnp.float32),
            pltpu.VMEM((E, OC), jnp.float32),
        ],
    )(obs, idx2, Wa1, ba1, Wa2, ba2, Wc1, bc1, Wc2, bc2)

    return (out[:, :A], out[:, A])


# speedup vs baseline: 1.1649x; 1.1649x over previous
import jax
import jax.numpy as jnp
import numpy as np
from jax import lax
from jax.experimental import pallas as pl
from jax.experimental.pallas import tpu as pltpu

B = 2048
D = 1024
E = 8
A = 16
H = 64
HC = 2 * H
OC = 32
EH = E * H
BM = 256


def _body(obs_ref, idx_ref, wa_ref, wc_ref, ba1_ref, wa2_ref, ba2_ref,
          bc1_ref, wc2_ref, bc2_ref, out_ref, b1_scr, w2_scr, b2_scr):
    @pl.when(pl.program_id(0) == 0)
    def _prep():
        w2_scr[...] = jnp.zeros((E, HC, OC), jnp.float32)
        b2_scr[...] = jnp.zeros((E, OC), jnp.float32)
        for e in range(E):
            b1_scr[e:e + 1, :H] = ba1_ref[e:e + 1]
            b1_scr[e:e + 1, H:] = bc1_ref[e:e + 1]
            w2_scr[e, :H, :A] = wa2_ref[e]
            w2_scr[e, H:, A:A + 1] = wc2_ref[e]
            b2_scr[e:e + 1, :A] = ba2_ref[e:e + 1]
            b2_scr[e:e + 1, A:A + 1] = bc2_ref[e:e + 1]

    x = obs_ref[...].astype(jnp.bfloat16)
    idx = idx_ref[...]
    ha_all = lax.dot_general(x, wa_ref[...], (((1,), (0,)), ((), ())),
                             preferred_element_type=jnp.float32)
    hc_all = lax.dot_general(x, wc_ref[...], (((1,), (0,)), ((), ())),
                             preferred_element_type=jnp.float32)
    pre = jnp.zeros((BM, HC), jnp.float32)
    for e in range(E):
        he = jnp.concatenate(
            [ha_all[:, e * H:(e + 1) * H], hc_all[:, e * H:(e + 1) * H]],
            axis=1)
        pre = jnp.where(idx == e, he, pre)
    b1 = jnp.zeros((BM, HC), jnp.float32)
    for e in range(E):
        b1 = jnp.where(idx == e, b1_scr[e][None, :], b1)
    h = jnp.tanh(pre + b1)
    acc = jnp.zeros((BM, OC), jnp.float32)
    for e in range(E):
        oe = lax.dot_general(h, w2_scr[e], (((1,), (0,)), ((), ())),
                             preferred_element_type=jnp.float32)
        acc = jnp.where(idx == e, oe + b2_scr[e][None, :], acc)
    out_ref[...] = acc


@jax.jit
def kernel(obs, idxs, Wa1, ba1, Wa2, ba2, Wc1, bc1, Wc2, bc2):
    bf = jnp.bfloat16
    wa_t = jnp.swapaxes(Wa1.astype(bf), 0, 1).reshape(D, EH)
    wc_t = jnp.swapaxes(Wc1.astype(bf), 0, 1).reshape(D, EH)
    idx2 = idxs.astype(jnp.int32).reshape(B, 1)

    def full(shape):
        return pl.BlockSpec(shape, lambda i: tuple(0 for _ in shape))

    out = pl.pallas_call(
        _body,
        grid=(B // BM,),
        in_specs=[
            pl.BlockSpec((BM, D), lambda i: (i, 0)),
            pl.BlockSpec((BM, 1), lambda i: (i, 0)),
            full((D, EH)),
            full((D, EH)),
            full((E, H)),
            full((E, H, A)),
            full((E, A)),
            full((E, H)),
            full((E, H, 1)),
            full((E, 1)),
        ],
        out_specs=pl.BlockSpec((BM, OC), lambda i: (i, 0)),
        out_shape=jax.ShapeDtypeStruct((B, OC), jnp.float32),
        scratch_shapes=[
            pltpu.VMEM((E, HC), jnp.float32),
            pltpu.VMEM((E, HC, OC), jnp.float32),
            pltpu.VMEM((E, OC), jnp.float32),
        ],
    )(obs, idx2, wa_t, wc_t, ba1, Wa2, ba2, bc1, Wc2, bc2)

    return (out[:, :A], out[:, A])


# speedup vs baseline: 1.3255x; 1.1378x over previous
import jax
import jax.numpy as jnp
import numpy as np
from jax import lax
from jax.experimental import pallas as pl
from jax.experimental.pallas import tpu as pltpu

B = 2048
D = 1024
E = 8
A = 16
H = 64
HC = 2 * H
OC = 32
EH = E * H
BM = 256


def _body(obs_ref, idx_ref, w_ref, ba1_ref, wa2_ref, ba2_ref,
          bc1_ref, wc2_ref, bc2_ref, out_ref, b1_scr, w2_scr, b2_scr):
    @pl.when(pl.program_id(0) == 0)
    def _prep():
        w2_scr[...] = jnp.zeros((E, HC, OC), jnp.float32)
        b2_scr[...] = jnp.zeros((E, OC), jnp.float32)
        for e in range(E):
            b1_scr[e:e + 1, :H] = ba1_ref[e:e + 1]
            b1_scr[e:e + 1, H:] = bc1_ref[e:e + 1]
            w2_scr[e, :H, :A] = wa2_ref[e]
            w2_scr[e, H:, A:A + 1] = wc2_ref[e]
            b2_scr[e:e + 1, :A] = ba2_ref[e:e + 1]
            b2_scr[e:e + 1, A:A + 1] = bc2_ref[e:e + 1]

    x = obs_ref[...].astype(jnp.bfloat16)
    idx = idx_ref[...]
    hall = lax.dot_general(x, w_ref[...], (((1,), (0,)), ((), ())),
                           preferred_element_type=jnp.float32)
    pre = jnp.zeros((BM, HC), jnp.float32)
    for e in range(E):
        he = hall[:, e * HC:(e + 1) * HC] + b1_scr[e][None, :]
        pre = jnp.where(idx == e, he, pre)
    h = jnp.tanh(pre)
    acc = jnp.zeros((BM, OC), jnp.float32)
    for e in range(E):
        oe = lax.dot_general(h, w2_scr[e], (((1,), (0,)), ((), ())),
                             preferred_element_type=jnp.float32)
        acc = jnp.where(idx == e, oe + b2_scr[e][None, :], acc)
    out_ref[...] = acc


@jax.jit
def kernel(obs, idxs, Wa1, ba1, Wa2, ba2, Wc1, bc1, Wc2, bc2):
    bf = jnp.bfloat16
    w_t = jnp.swapaxes(
        jnp.concatenate([Wa1.astype(bf), Wc1.astype(bf)], axis=2),
        0, 1).reshape(D, E * HC)
    idx2 = idxs.astype(jnp.int32).reshape(B, 1)

    def full(shape):
        return pl.BlockSpec(shape, lambda i: tuple(0 for _ in shape))

    out = pl.pallas_call(
        _body,
        grid=(B // BM,),
        in_specs=[
            pl.BlockSpec((BM, D), lambda i: (i, 0)),
            pl.BlockSpec((BM, 1), lambda i: (i, 0)),
            full((D, E * HC)),
            full((E, H)),
            full((E, H, A)),
            full((E, A)),
            full((E, H)),
            full((E, H, 1)),
            full((E, 1)),
        ],
        out_specs=pl.BlockSpec((BM, OC), lambda i: (i, 0)),
        out_shape=jax.ShapeDtypeStruct((B, OC), jnp.float32),
        scratch_shapes=[
            pltpu.VMEM((E, HC), jnp.float32),
            pltpu.VMEM((E, HC, OC), jnp.float32),
            pltpu.VMEM((E, OC), jnp.float32),
        ],
    )(obs, idx2, w_t, ba1, Wa2, ba2, bc1, Wc2, bc2)

    return (out[:, :A], out[:, A])


# speedup vs baseline: 1.4278x; 1.0771x over previous
import jax
import jax.numpy as jnp
import numpy as np
from jax import lax
from jax.experimental import pallas as pl
from jax.experimental.pallas import tpu as pltpu

B = 2048
D = 1024
E = 8
A = 16
H = 64
HC = 2 * H
OC = 32
EH = E * H
BM = 512


def _body(obs_ref, idx_ref, w_ref, ba1_ref, wa2_ref, ba2_ref,
          bc1_ref, wc2_ref, bc2_ref, out_ref, b1_scr, w2_scr, b2_scr):
    @pl.when(pl.program_id(0) == 0)
    def _prep():
        w2_scr[...] = jnp.zeros((E, HC, OC), jnp.float32)
        b2_scr[...] = jnp.zeros((E, OC), jnp.float32)
        for e in range(E):
            b1_scr[e:e + 1, :H] = ba1_ref[e:e + 1]
            b1_scr[e:e + 1, H:] = bc1_ref[e:e + 1]
            w2_scr[e, :H, :A] = wa2_ref[e]
            w2_scr[e, H:, A:A + 1] = wc2_ref[e]
            b2_scr[e:e + 1, :A] = ba2_ref[e:e + 1]
            b2_scr[e:e + 1, A:A + 1] = bc2_ref[e:e + 1]

    x = obs_ref[...].astype(jnp.bfloat16)
    idx = idx_ref[...]
    hall = lax.dot_general(x, w_ref[...], (((1,), (0,)), ((), ())),
                           preferred_element_type=jnp.float32)
    pre = jnp.zeros((BM, HC), jnp.float32)
    for e in range(E):
        he = hall[:, e * HC:(e + 1) * HC] + b1_scr[e][None, :]
        pre = jnp.where(idx == e, he, pre)
    h = jnp.tanh(pre)
    acc = jnp.zeros((BM, OC), jnp.float32)
    for e in range(E):
        oe = lax.dot_general(h, w2_scr[e], (((1,), (0,)), ((), ())),
                             preferred_element_type=jnp.float32)
        acc = jnp.where(idx == e, oe + b2_scr[e][None, :], acc)
    out_ref[...] = acc


@jax.jit
def kernel(obs, idxs, Wa1, ba1, Wa2, ba2, Wc1, bc1, Wc2, bc2):
    bf = jnp.bfloat16
    w_t = jnp.swapaxes(
        jnp.concatenate([Wa1.astype(bf), Wc1.astype(bf)], axis=2),
        0, 1).reshape(D, E * HC)
    idx2 = idxs.astype(jnp.int32).reshape(B, 1)

    def full(shape):
        return pl.BlockSpec(shape, lambda i: tuple(0 for _ in shape))

    out = pl.pallas_call(
        _body,
        grid=(B // BM,),
        in_specs=[
            pl.BlockSpec((BM, D), lambda i: (i, 0)),
            pl.BlockSpec((BM, 1), lambda i: (i, 0)),
            full((D, E * HC)),
            full((E, H)),
            full((E, H, A)),
            full((E, A)),
            full((E, H)),
            full((E, H, 1)),
            full((E, 1)),
        ],
        out_specs=pl.BlockSpec((BM, OC), lambda i: (i, 0)),
        out_shape=jax.ShapeDtypeStruct((B, OC), jnp.float32),
        scratch_shapes=[
            pltpu.VMEM((E, HC), jnp.float32),
            pltpu.VMEM((E, HC, OC), jnp.float32),
            pltpu.VMEM((E, OC), jnp.float32),
        ],
    )(obs, idx2, w_t, ba1, Wa2, ba2, bc1, Wc2, bc2)

    return (out[:, :A], out[:, A])


# speedup vs baseline: 1.4314x; 1.0025x over previous
import jax
import jax.numpy as jnp
import numpy as np
from jax import lax
from jax.experimental import pallas as pl
from jax.experimental.pallas import tpu as pltpu

B = 2048
D = 1024
E = 8
A = 16
H = 64
HC = 2 * H
OC = 32
EH = E * H
BM = 1024


def _body(obs_ref, idx_ref, w_ref, ba1_ref, wa2_ref, ba2_ref,
          bc1_ref, wc2_ref, bc2_ref, out_ref, b1_scr, w2_scr, b2_scr):
    @pl.when(pl.program_id(0) == 0)
    def _prep():
        w2_scr[...] = jnp.zeros((E, HC, OC), jnp.float32)
        b2_scr[...] = jnp.zeros((E, OC), jnp.float32)
        for e in range(E):
            b1_scr[e:e + 1, :H] = ba1_ref[e:e + 1]
            b1_scr[e:e + 1, H:] = bc1_ref[e:e + 1]
            w2_scr[e, :H, :A] = wa2_ref[e]
            w2_scr[e, H:, A:A + 1] = wc2_ref[e]
            b2_scr[e:e + 1, :A] = ba2_ref[e:e + 1]
            b2_scr[e:e + 1, A:A + 1] = bc2_ref[e:e + 1]

    x = obs_ref[...].astype(jnp.bfloat16)
    idx = idx_ref[...]
    hall = lax.dot_general(x, w_ref[...], (((1,), (0,)), ((), ())),
                           preferred_element_type=jnp.float32)
    pre = jnp.zeros((BM, HC), jnp.float32)
    for e in range(E):
        he = hall[:, e * HC:(e + 1) * HC] + b1_scr[e][None, :]
        pre = jnp.where(idx == e, he, pre)
    h = jnp.tanh(pre)
    acc = jnp.zeros((BM, OC), jnp.float32)
    for e in range(E):
        oe = lax.dot_general(h, w2_scr[e], (((1,), (0,)), ((), ())),
                             preferred_element_type=jnp.float32)
        acc = jnp.where(idx == e, oe + b2_scr[e][None, :], acc)
    out_ref[...] = acc


@jax.jit
def kernel(obs, idxs, Wa1, ba1, Wa2, ba2, Wc1, bc1, Wc2, bc2):
    bf = jnp.bfloat16
    w_t = jnp.swapaxes(
        jnp.concatenate([Wa1.astype(bf), Wc1.astype(bf)], axis=2),
        0, 1).reshape(D, E * HC)
    idx2 = idxs.astype(jnp.int32).reshape(B, 1)

    def full(shape):
        return pl.BlockSpec(shape, lambda i: tuple(0 for _ in shape))

    out = pl.pallas_call(
        _body,
        grid=(B // BM,),
        in_specs=[
            pl.BlockSpec((BM, D), lambda i: (i, 0)),
            pl.BlockSpec((BM, 1), lambda i: (i, 0)),
            full((D, E * HC)),
            full((E, H)),
            full((E, H, A)),
            full((E, A)),
            full((E, H)),
            full((E, H, 1)),
            full((E, 1)),
        ],
        out_specs=pl.BlockSpec((BM, OC), lambda i: (i, 0)),
        out_shape=jax.ShapeDtypeStruct((B, OC), jnp.float32),
        scratch_shapes=[
            pltpu.VMEM((E, HC), jnp.float32),
            pltpu.VMEM((E, HC, OC), jnp.float32),
            pltpu.VMEM((E, OC), jnp.float32),
        ],
    )(obs, idx2, w_t, ba1, Wa2, ba2, bc1, Wc2, bc2)

    return (out[:, :A], out[:, A])
